# baseline (speedup 1.0000x reference)
import functools

import jax
import jax.numpy as jnp
from jax import lax
from jax.experimental import pallas as pl
from jax.experimental.pallas import tpu as pltpu
from jax.experimental.pallas import tpu_sc as plsc

B, S, D = 2, 2048, 768
E, K = 8, 2
H = 4 * D
T = B * S
CAP = int(1.25 * T * K / E)
M = 128
NPAD = 9216
NB = NPAD // M

NW = 32
TPW = T // NW
SPW = NPAD // NW
LANES = 16

_f32 = jnp.float32
_i32 = jnp.int32


def _router_body(x_ref, wr_ref, g_ref, b_ref,
                 normed_ref, pos0_ref, pos1_ref, wa_ref, wb_ref, be_ref,
                 bv_ref, loss_ref,
                 oh_scr, pref_scr, rank_scr):
    xv = x_ref[...]
    mu = jnp.mean(xv, axis=1, keepdims=True)
    xc = xv - mu
    var = jnp.mean(xc * xc, axis=1, keepdims=True)
    normed = xc / jnp.sqrt(var + 1e-5) * g_ref[...] + b_ref[...]
    normed_ref[...] = normed

    logits = lax.dot_general(normed, wr_ref[...],
                             (((1,), (1,)), ((), ())),
                             preferred_element_type=_f32)
    mx = jnp.max(logits, axis=1, keepdims=True)
    ex = jnp.exp(logits - mx)
    rw = ex / jnp.sum(ex, axis=1, keepdims=True)

    eids = lax.broadcasted_iota(_i32, (T, E), 1)
    m1 = jnp.max(rw, axis=1, keepdims=True)
    i1 = jnp.min(jnp.where(rw == m1, eids, E), axis=1, keepdims=True)
    oh0 = (eids == i1)
    rw2 = jnp.where(oh0, -jnp.inf, rw)
    m2 = jnp.max(rw2, axis=1, keepdims=True)
    i2 = jnp.min(jnp.where(rw2 == m2, eids, E), axis=1, keepdims=True)
    oh1 = (eids == i2)

    oh0f = oh0.astype(_f32)
    oh1f = oh1.astype(_f32)
    h1 = jnp.sum(oh0f, axis=0, keepdims=True)
    h2 = jnp.sum(oh1f, axis=0, keepdims=True)
    acc2 = jnp.sum(jnp.where(oh1, (h1 < CAP).astype(_f32), 0.0),
                   axis=1, keepdims=True)

    s1 = m1 + m2 + 1e-8
    tn0 = m1 / s1
    tn1 = m2 / s1
    s2 = tn0 + acc2 * tn1 + 1e-8
    wa = tn0 / s2
    wb = acc2 * tn1 / s2
    wa_ref[...] = wa
    wb_ref[...] = wb

    z = jnp.mean(logits * logits)
    ec = jnp.sum(wa * oh0f + wb * oh1f, axis=0, keepdims=True)
    lb = jnp.mean((ec / T - (K / E)) ** 2)
    loss_ref[...] = jnp.reshape(0.001 * z + 0.001 * lb, (1, 1))

    sizes = h1 + h2
    padded = jnp.floor((sizes + (M - 1)) / M) * M
    r8 = lax.broadcasted_iota(_i32, (E, E), 0)
    c8 = lax.broadcasted_iota(_i32, (E, E), 1)
    lt8 = (r8 < c8).astype(_f32)
    base = lax.dot_general(padded, lt8, (((1,), (0,)), ((), ())),
                           preferred_element_type=_f32)

    rr = lax.broadcasted_iota(_i32, (128, 128), 0)
    cc = lax.broadcasted_iota(_i32, (128, 128), 1)
    ltri = (rr > cc).astype(_f32)
    nchunks = T // 128

    def _excl_cumsum(ohf):
        oh_scr[...] = ohf

        def body1(c, acc):
            pref_scr[pl.ds(c, 1), :] = acc
            blk = oh_scr[pl.ds(c * 128, 128), :]
            return acc + jnp.sum(blk, axis=0, keepdims=True)

        lax.fori_loop(0, nchunks, body1, jnp.zeros((1, E), _f32))

        def body2(c, _):
            blk = oh_scr[pl.ds(c * 128, 128), :]
            within = jnp.dot(ltri, blk, preferred_element_type=_f32)
            rank_scr[pl.ds(c * 128, 128), :] = within + pref_scr[pl.ds(c, 1), :]
            return 0

        lax.fori_loop(0, nchunks, body2, 0)
        return rank_scr[...]

    rank0 = _excl_cumsum(oh0f)
    pos0 = jnp.sum(oh0f * base, axis=1, keepdims=True) + \
        jnp.sum(oh0f * rank0, axis=1, keepdims=True)
    pos0_ref[...] = pos0.astype(_i32)

    rank1 = _excl_cumsum(oh1f)
    pos1 = jnp.sum(oh1f * (base + h1), axis=1, keepdims=True) + \
        jnp.sum(oh1f * rank1, axis=1, keepdims=True)
    pos1_ref[...] = pos1.astype(_i32)

    ends = base + padded
    sb = (lax.broadcasted_iota(_i32, (NB, E), 0) * M).astype(_f32)
    be = jnp.sum((sb >= ends).astype(_i32), axis=1, keepdims=True)
    be_ref[...] = jnp.minimum(be, E - 1)
    total = jnp.max(ends, axis=1, keepdims=True)
    bv_ref[...] = (sb[:, :1] < total).astype(_i32)


_router_call = pl.pallas_call(
    _router_body,
    out_shape=[
        jax.ShapeDtypeStruct((T, D), _f32),
        jax.ShapeDtypeStruct((T, 1), _i32),
        jax.ShapeDtypeStruct((T, 1), _i32),
        jax.ShapeDtypeStruct((T, 1), _f32),
        jax.ShapeDtypeStruct((T, 1), _f32),
        jax.ShapeDtypeStruct((NB, 1), _i32),
        jax.ShapeDtypeStruct((NB, 1), _i32),
        jax.ShapeDtypeStruct((1, 1), _f32),
    ],
    scratch_shapes=[
        pltpu.VMEM((T, E), _f32),
        pltpu.VMEM((T // 128, E), _f32),
        pltpu.VMEM((T, E), _f32),
    ],
)


def _dispatch_body(normed_hbm, pos0_hbm, pos1_hbm, wa_hbm, wb_hbm,
                   xg_hbm, rw_hbm,
                   rows_v, p0_v, p1_v, wa_v, wb_v, sem):
    wid = lax.axis_index("s") * 2 + lax.axis_index("c")
    tok0 = wid * TPW
    pltpu.sync_copy(pos0_hbm.at[pl.ds(tok0, TPW)], p0_v)
    pltpu.sync_copy(pos1_hbm.at[pl.ds(tok0, TPW)], p1_v)
    pltpu.sync_copy(wa_hbm.at[pl.ds(tok0, TPW)], wa_v)
    pltpu.sync_copy(wb_hbm.at[pl.ds(tok0, TPW)], wb_v)
    pltpu.sync_copy(normed_hbm.at[pl.ds(tok0, TPW)], rows_v)
    c1 = pltpu.async_copy(rows_v, xg_hbm.at[p0_v], sem)
    c2 = pltpu.async_copy(rows_v, xg_hbm.at[p1_v], sem)
    c3 = pltpu.async_copy(wa_v, rw_hbm.at[p0_v], sem)
    c4 = pltpu.async_copy(wb_v, rw_hbm.at[p1_v], sem)
    c1.wait()
    c2.wait()
    c3.wait()
    c4.wait()


_dispatch_call = functools.partial(
    pl.kernel,
    out_type=[
        jax.ShapeDtypeStruct((NPAD, D), _f32),
        jax.ShapeDtypeStruct((NPAD,), _f32),
    ],
    mesh=plsc.VectorSubcoreMesh(core_axis_name="c", subcore_axis_name="s"),
    scratch_types=[
        pltpu.VMEM((TPW, D), _f32),
        pltpu.VMEM((TPW,), _i32),
        pltpu.VMEM((TPW,), _i32),
        pltpu.VMEM((TPW,), _f32),
        pltpu.VMEM((TPW,), _f32),
        pltpu.SemaphoreType.DMA,
    ],
)(_dispatch_body)


def _gmm_body(be_ref, bv_ref, xg_ref, w1_ref, w2_ref, wd_ref, rw_ref, out_ref):
    b = pl.program_id(0)

    @pl.when(bv_ref[b] != 0)
    def _():
        xb = xg_ref[...].astype(jnp.bfloat16)
        x1 = lax.dot_general(xb, w1_ref[0], (((1,), (1,)), ((), ())),
                             preferred_element_type=_f32)
        x2 = lax.dot_general(xb, w2_ref[0], (((1,), (1,)), ((), ())),
                             preferred_element_type=_f32)
        hid = (x1 * (x2 / (1.0 + jnp.exp(-x2)))).astype(jnp.bfloat16)
        res = lax.dot_general(hid, wd_ref[0], (((1,), (1,)), ((), ())),
                              preferred_element_type=_f32)
        out_ref[...] = res * rw_ref[...]


_gmm_call = pl.pallas_call(
    _gmm_body,
    grid_spec=pltpu.PrefetchScalarGridSpec(
        num_scalar_prefetch=2,
        grid=(NB,),
        in_specs=[
            pl.BlockSpec((M, D), lambda b, be, bv: (b, 0)),
            pl.BlockSpec((1, H, D), lambda b, be, bv: (be[b], 0, 0)),
            pl.BlockSpec((1, H, D), lambda b, be, bv: (be[b], 1, 0)),
            pl.BlockSpec((1, D, H), lambda b, be, bv: (be[b], 0, 0)),
            pl.BlockSpec((M, 1), lambda b, be, bv: (b, 0)),
        ],
        out_specs=pl.BlockSpec((M, D), lambda b, be, bv: (b, 0)),
    ),
    out_shape=jax.ShapeDtypeStruct((NPAD, D), _f32),
)


def _combine_body(outg_hbm, pos0_hbm, pos1_hbm, out_hbm,
                  a_v, b_v, p0_v, p1_v, sem):
    wid = lax.axis_index("s") * 2 + lax.axis_index("c")
    csz = 64
    for ci in range(TPW // csz):
        tb = wid * TPW + ci * csz
        pltpu.sync_copy(pos0_hbm.at[pl.ds(tb, csz)], p0_v)
        pltpu.sync_copy(pos1_hbm.at[pl.ds(tb, csz)], p1_v)
        ca = pltpu.async_copy(outg_hbm.at[p0_v], a_v, sem)
        cb = pltpu.async_copy(outg_hbm.at[p1_v], b_v, sem)
        ca.wait()
        cb.wait()

        def row_body(r, _):
            for jv in range(D // LANES):
                sl = pl.ds(jv * LANES, LANES)
                a_v[r, sl] = a_v[r, sl] + b_v[r, sl]
            return 0

        lax.fori_loop(0, csz, row_body, 0)
        pltpu.sync_copy(a_v, out_hbm.at[pl.ds(tb, csz)])


_combine_call = functools.partial(
    pl.kernel,
    out_type=jax.ShapeDtypeStruct((T, D), _f32),
    mesh=plsc.VectorSubcoreMesh(core_axis_name="c", subcore_axis_name="s"),
    scratch_types=[
        pltpu.VMEM((64, D), _f32),
        pltpu.VMEM((64, D), _f32),
        pltpu.VMEM((64,), _i32),
        pltpu.VMEM((64,), _i32),
        pltpu.SemaphoreType.DMA,
    ],
)(_combine_body)


def kernel(x, Wr, Wgu, Wd, gamma, beta):
    flat = x.reshape(T, D)
    normed, pos0, pos1, wa, wb, be, bv, rloss = _router_call(
        flat, Wr, gamma.reshape(1, D), beta.reshape(1, D))
    xg, rwv = _dispatch_call(normed, pos0.reshape(T), pos1.reshape(T),
                             wa.reshape(T), wb.reshape(T))
    wgu_b = Wgu.astype(jnp.bfloat16)
    outg = _gmm_call(be.reshape(NB), bv.reshape(NB), xg, wgu_b, wgu_b,
                     Wd.astype(jnp.bfloat16), rwv.reshape(NPAD, 1))
    comb = _combine_call(outg, pos0.reshape(T), pos1.reshape(T))
    return comb.reshape(B, S, D), rloss[0, 0]


# speedup vs baseline: 1.0931x; 1.0931x over previous
import functools

import jax
import jax.numpy as jnp
from jax import lax
from jax.experimental import pallas as pl
from jax.experimental.pallas import tpu as pltpu
from jax.experimental.pallas import tpu_sc as plsc

B, S, D = 2, 2048, 768
E, K = 8, 2
H = 4 * D
T = B * S
CAP = int(1.25 * T * K / E)
M = 128
NPAD = 9216
NB = NPAD // M

NW = 32
TPW = T // NW
SPW = NPAD // NW
LANES = 16

_f32 = jnp.float32
_i32 = jnp.int32


def _router_body(x_ref, wr_ref, g_ref, b_ref,
                 normed_ref, pos0_ref, pos1_ref, wa_ref, wb_ref, be_ref,
                 bv_ref, loss_ref,
                 oh_scr, rank_scr):
    xv = x_ref[...]
    mu = jnp.mean(xv, axis=1, keepdims=True)
    xc = xv - mu
    var = jnp.mean(xc * xc, axis=1, keepdims=True)
    normed = xc / jnp.sqrt(var + 1e-5) * g_ref[...] + b_ref[...]
    normed_ref[...] = normed

    logits = lax.dot_general(normed, wr_ref[...],
                             (((1,), (1,)), ((), ())),
                             preferred_element_type=_f32)
    mx = jnp.max(logits, axis=1, keepdims=True)
    ex = jnp.exp(logits - mx)
    rw = ex / jnp.sum(ex, axis=1, keepdims=True)

    eids = lax.broadcasted_iota(_i32, (T, E), 1)
    m1 = jnp.max(rw, axis=1, keepdims=True)
    i1 = jnp.min(jnp.where(rw == m1, eids, E), axis=1, keepdims=True)
    oh0 = (eids == i1)
    rw2 = jnp.where(oh0, -jnp.inf, rw)
    m2 = jnp.max(rw2, axis=1, keepdims=True)
    i2 = jnp.min(jnp.where(rw2 == m2, eids, E), axis=1, keepdims=True)
    oh1 = (eids == i2)

    oh0f = oh0.astype(_f32)
    oh1f = oh1.astype(_f32)
    h1 = jnp.sum(oh0f, axis=0, keepdims=True)
    h2 = jnp.sum(oh1f, axis=0, keepdims=True)
    acc2 = jnp.sum(jnp.where(oh1, (h1 < CAP).astype(_f32), 0.0),
                   axis=1, keepdims=True)

    s1 = m1 + m2 + 1e-8
    tn0 = m1 / s1
    tn1 = m2 / s1
    s2 = tn0 + acc2 * tn1 + 1e-8
    wa = tn0 / s2
    wb = acc2 * tn1 / s2
    wa_ref[...] = wa
    wb_ref[...] = wb

    z = jnp.mean(logits * logits)
    ec = jnp.sum(wa * oh0f + wb * oh1f, axis=0, keepdims=True)
    lb = jnp.mean((ec / T - (K / E)) ** 2)
    loss_ref[...] = jnp.reshape(0.001 * z + 0.001 * lb, (1, 1))

    sizes = h1 + h2
    padded = jnp.floor((sizes + (M - 1)) / M) * M
    r8 = lax.broadcasted_iota(_i32, (E, E), 0)
    c8 = lax.broadcasted_iota(_i32, (E, E), 1)
    lt8 = (r8 < c8).astype(_f32)
    base = lax.dot_general(padded, lt8, (((1,), (0,)), ((), ())),
                           preferred_element_type=_f32)

    CH = 512
    rr = lax.broadcasted_iota(_i32, (CH, CH), 0)
    cc = lax.broadcasted_iota(_i32, (CH, CH), 1)
    ltri = (rr > cc).astype(_f32)
    oh_scr[...] = jnp.concatenate([oh0f, oh1f], axis=1)

    def body(c, acc):
        blk = oh_scr[pl.ds(c * CH, CH), :]
        within = jnp.dot(ltri, blk, preferred_element_type=_f32)
        rank_scr[pl.ds(c * CH, CH), :] = within + acc
        return acc + jnp.sum(blk, axis=0, keepdims=True)

    lax.fori_loop(0, T // CH, body, jnp.zeros((1, 2 * E), _f32))

    rank = rank_scr[...]
    pos0 = jnp.sum(oh0f * (base + rank[:, :E]), axis=1, keepdims=True)
    pos0_ref[...] = pos0.astype(_i32)
    pos1 = jnp.sum(oh1f * (base + h1 + rank[:, E:]), axis=1, keepdims=True)
    pos1_ref[...] = pos1.astype(_i32)

    ends = base + padded
    sb = (lax.broadcasted_iota(_i32, (NB, E), 0) * M).astype(_f32)
    be = jnp.sum((sb >= ends).astype(_i32), axis=1, keepdims=True)
    be_ref[...] = jnp.minimum(be, E - 1)
    total = jnp.max(ends, axis=1, keepdims=True)
    bv_ref[...] = (sb[:, :1] < total).astype(_i32)


_router_call = pl.pallas_call(
    _router_body,
    out_shape=[
        jax.ShapeDtypeStruct((T, D), _f32),
        jax.ShapeDtypeStruct((T, 1), _i32),
        jax.ShapeDtypeStruct((T, 1), _i32),
        jax.ShapeDtypeStruct((T, 1), _f32),
        jax.ShapeDtypeStruct((T, 1), _f32),
        jax.ShapeDtypeStruct((NB, 1), _i32),
        jax.ShapeDtypeStruct((NB, 1), _i32),
        jax.ShapeDtypeStruct((1, 1), _f32),
    ],
    scratch_shapes=[
        pltpu.VMEM((T, 2 * E), _f32),
        pltpu.VMEM((T, 2 * E), _f32),
    ],
)


def _dispatch_body(normed_hbm, pos0_hbm, pos1_hbm, wa_hbm, wb_hbm,
                   xg_hbm, rw_hbm,
                   rows_v, p0_v, p1_v, wa_v, wb_v, sem):
    wid = lax.axis_index("s") * 2 + lax.axis_index("c")
    tok0 = wid * TPW
    pltpu.sync_copy(pos0_hbm.at[pl.ds(tok0, TPW)], p0_v)
    pltpu.sync_copy(pos1_hbm.at[pl.ds(tok0, TPW)], p1_v)
    pltpu.sync_copy(wa_hbm.at[pl.ds(tok0, TPW)], wa_v)
    pltpu.sync_copy(wb_hbm.at[pl.ds(tok0, TPW)], wb_v)
    pltpu.sync_copy(normed_hbm.at[pl.ds(tok0, TPW)], rows_v)
    c1 = pltpu.async_copy(rows_v, xg_hbm.at[p0_v], sem)
    c2 = pltpu.async_copy(rows_v, xg_hbm.at[p1_v], sem)
    c3 = pltpu.async_copy(wa_v, rw_hbm.at[p0_v], sem)
    c4 = pltpu.async_copy(wb_v, rw_hbm.at[p1_v], sem)
    c1.wait()
    c2.wait()
    c3.wait()
    c4.wait()


_dispatch_call = functools.partial(
    pl.kernel,
    out_type=[
        jax.ShapeDtypeStruct((NPAD, D), _f32),
        jax.ShapeDtypeStruct((NPAD,), _f32),
    ],
    mesh=plsc.VectorSubcoreMesh(core_axis_name="c", subcore_axis_name="s"),
    scratch_types=[
        pltpu.VMEM((TPW, D), _f32),
        pltpu.VMEM((TPW,), _i32),
        pltpu.VMEM((TPW,), _i32),
        pltpu.VMEM((TPW,), _f32),
        pltpu.VMEM((TPW,), _f32),
        pltpu.SemaphoreType.DMA,
    ],
)(_dispatch_body)


def _gmm_body(be_ref, bv_ref, xg_ref, w1_ref, w2_ref, wd_ref, rw_ref, out_ref):
    b = pl.program_id(0)

    @pl.when(bv_ref[b] != 0)
    def _():
        xb = xg_ref[...]
        x1 = lax.dot_general(xb, w1_ref[0], (((1,), (1,)), ((), ())),
                             preferred_element_type=_f32)
        x2 = lax.dot_general(xb, w2_ref[0], (((1,), (1,)), ((), ())),
                             preferred_element_type=_f32)
        hid = x1 * (x2 / (1.0 + jnp.exp(-x2)))
        res = lax.dot_general(hid, wd_ref[0], (((1,), (1,)), ((), ())),
                              preferred_element_type=_f32)
        out_ref[...] = res * rw_ref[...]


_gmm_call = pl.pallas_call(
    _gmm_body,
    grid_spec=pltpu.PrefetchScalarGridSpec(
        num_scalar_prefetch=2,
        grid=(NB,),
        in_specs=[
            pl.BlockSpec((M, D), lambda b, be, bv: (b, 0)),
            pl.BlockSpec((1, H, D), lambda b, be, bv: (be[b], 0, 0)),
            pl.BlockSpec((1, H, D), lambda b, be, bv: (be[b], 1, 0)),
            pl.BlockSpec((1, D, H), lambda b, be, bv: (be[b], 0, 0)),
            pl.BlockSpec((M, 1), lambda b, be, bv: (b, 0)),
        ],
        out_specs=pl.BlockSpec((M, D), lambda b, be, bv: (b, 0)),
    ),
    out_shape=jax.ShapeDtypeStruct((NPAD, D), _f32),
)


def _combine_body(outg_hbm, pos0_hbm, pos1_hbm, out_hbm,
                  a_v, b_v, p0_v, p1_v, sem):
    wid = lax.axis_index("s") * 2 + lax.axis_index("c")
    csz = 64
    for ci in range(TPW // csz):
        tb = wid * TPW + ci * csz
        pltpu.sync_copy(pos0_hbm.at[pl.ds(tb, csz)], p0_v)
        pltpu.sync_copy(pos1_hbm.at[pl.ds(tb, csz)], p1_v)
        ca = pltpu.async_copy(outg_hbm.at[p0_v], a_v, sem)
        cb = pltpu.async_copy(outg_hbm.at[p1_v], b_v, sem)
        ca.wait()
        cb.wait()

        def row_body(r, _):
            for jv in range(D // LANES):
                sl = pl.ds(jv * LANES, LANES)
                a_v[r, sl] = a_v[r, sl] + b_v[r, sl]
            return 0

        lax.fori_loop(0, csz, row_body, 0)
        pltpu.sync_copy(a_v, out_hbm.at[pl.ds(tb, csz)])


_combine_call = functools.partial(
    pl.kernel,
    out_type=jax.ShapeDtypeStruct((T, D), _f32),
    mesh=plsc.VectorSubcoreMesh(core_axis_name="c", subcore_axis_name="s"),
    scratch_types=[
        pltpu.VMEM((64, D), _f32),
        pltpu.VMEM((64, D), _f32),
        pltpu.VMEM((64,), _i32),
        pltpu.VMEM((64,), _i32),
        pltpu.SemaphoreType.DMA,
    ],
)(_combine_body)


def kernel(x, Wr, Wgu, Wd, gamma, beta):
    flat = x.reshape(T, D)
    normed, pos0, pos1, wa, wb, be, bv, rloss = _router_call(
        flat, Wr, gamma.reshape(1, D), beta.reshape(1, D))
    xg, rwv = _dispatch_call(normed, pos0.reshape(T), pos1.reshape(T),
                             wa.reshape(T), wb.reshape(T))
    outg = _gmm_call(be.reshape(NB), bv.reshape(NB), xg, Wgu, Wgu, Wd,
                     rwv.reshape(NPAD, 1))
    comb = _combine_call(outg, pos0.reshape(T), pos1.reshape(T))
    return comb.reshape(B, S, D), rloss[0, 0]
